# Initial kernel scaffold; baseline (speedup 1.0000x reference)
import jax
import jax.numpy as jnp
from jax.experimental import pallas as pl


def kernel(x, edge_index, batch, bn0_g, bn0_b, We0, be0, Wc0, bc0, bn1_g, bn1_b, We1, be1, Wc1, bc1, bn2_g, bn2_b, We2, be2, Wc2, bc2, bnfc_g, bnfc_b, Wlin, blin, bnh_g, bnh_b, Wcls, bcls):
    raise NotImplementedError("write your pallas kernel here")


# speedup vs baseline: 8.6792x; 8.6792x over previous
import functools

import jax
import jax.numpy as jnp
from jax import lax
from jax.experimental import pallas as pl
from jax.experimental.pallas import tpu as pltpu
from jax.experimental.pallas import tpu_sc as plsc

N = 10000
E = 320000
D = 128
G = 64
C = 10

NSUB = 16
K = 128
CH = (E + NSUB * K - 1) // (NSUB * K)
EPAD = NSUB * K * CH
N1 = 10112
RPT = N1 // NSUB

BR = 1000
GRID = N // BR

_f32 = jnp.float32


def _sc_body(tpq, src2, dstp, zrows, out, idx_s, idx_d, rows, acc, sem):
    cid = lax.axis_index("c")
    sid = lax.axis_index("s")
    rbase = sid * RPT

    pltpu.sync_copy(zrows, rows.at[pl.ds(0, K)])
    for j in range((RPT + K - 1) // K):
        sz = min(K, RPT - j * K)
        pltpu.sync_copy(rows.at[pl.ds(0, sz)],
                        acc.at[pl.ds(rbase + j * K, sz)])
    plsc.subcore_barrier()

    ebase = sid * (CH * K)

    def chunk(i, carry):
        b = ebase + i * K
        pltpu.sync_copy(src2.at[pl.ds(cid * EPAD + b, K)], idx_s.at[0])
        pltpu.sync_copy(dstp.at[pl.ds(b, K)], idx_d.at[0])
        pltpu.async_copy(tpq.at[idx_s.at[0]], rows.at[pl.ds(0, K)], sem).wait()
        pltpu.sync_copy(rows.at[pl.ds(0, K)], acc.at[idx_d.at[0]], add=True)
        return carry

    lax.fori_loop(0, CH, chunk, 0)
    plsc.subcore_barrier()

    for j in range((RPT + K - 1) // K):
        sz = min(K, RPT - j * K)
        pltpu.sync_copy(acc.at[pl.ds(rbase + j * K, sz)],
                        rows.at[pl.ds(0, sz)])
        pltpu.sync_copy(rows.at[pl.ds(0, sz)],
                        out.at[pl.ds(cid * N1 + rbase + j * K, sz)])


@functools.cache
def _sc_kernel():
    return pl.kernel(
        _sc_body,
        out_type=jax.ShapeDtypeStruct((2 * N1, D), _f32),
        mesh=plsc.VectorSubcoreMesh(core_axis_name="c", subcore_axis_name="s"),
        scratch_types=[
            pltpu.VMEM((2, K), jnp.int32),
            pltpu.VMEM((2, K), jnp.int32),
            pltpu.VMEM((2 * K, D), _f32),
            pltpu.VMEM_SHARED((N1, D), _f32),
            pltpu.SemaphoreType.DMA,
        ],
    )


def _sc_edge_pass(tpq2n, src2, dstp, zrows):
    return _sc_kernel()(tpq2n, src2, dstp, zrows)


def _node_m(h, g, b, we, be):
    e = we[7:8, :] + be[...]
    m = jnp.maximum(h * g[...] + b[...] + e, 0.0) + 1e-7
    return m


def _tpq_body(h_ref, g_ref, b_ref, we_ref, be_ref, tpq_ref):
    m = _node_m(h_ref[...], g_ref, b_ref, we_ref, be_ref)
    p = jnp.exp(m)
    tpq_ref[0] = p
    tpq_ref[1] = m * p


def _conv_out(s_ref, h_ref, g0, b0, wc, bc):
    agg = s_ref[1] / (s_ref[0] + 1e-30)
    hn = h_ref[...] * g0[...] + b0[...]
    z = jnp.dot(hn + agg, wc[...], preferred_element_type=_f32) + bc[...]
    return jnp.maximum(z, 0.0)


def _ba_body(s_ref, h_ref, g0, b0, wc, bc, g1, b1, we1, be1, hout_ref, tpq_ref):
    hnew = _conv_out(s_ref, h_ref, g0, b0, wc, bc)
    hout_ref[...] = hnew
    m = _node_m(hnew, g1, b1, we1, be1)
    p = jnp.exp(m)
    tpq_ref[0] = p
    tpq_ref[1] = m * p


def _b3_body(s_ref, h_ref, g2, b2, wc2, bc2, batch_ref, gfc, bfc, wlin, blin,
             gh, bh, wcls, bcls, out_ref, pooled):
    i = pl.program_id(0)
    h3 = _conv_out(s_ref, h_ref, g2, b2, wc2, bc2)
    bvec = batch_ref[0, 0, :]
    onehot = (bvec[:, None]
              == lax.broadcasted_iota(jnp.int32, (BR, G), 1)).astype(_f32)
    part = lax.dot_general(onehot, h3, (((0,), (0,)), ((), ())),
                           preferred_element_type=_f32)

    @pl.when(i == 0)
    def _():
        pooled[...] = jnp.zeros_like(pooled)

    pooled[...] += part

    @pl.when(i == GRID - 1)
    def _():
        pool = pooled[...]
        z = jnp.maximum(
            jnp.dot(pool * gfc[...] + bfc[...], wlin[...],
                    preferred_element_type=_f32) + blin[...], 0.0)
        z = z * gh[...] + bh[...]
        logits = jnp.dot(z, wcls[...], preferred_element_type=_f32) + bcls[...]
        colid = lax.broadcasted_iota(jnp.int32, (G, D), 1)
        mask = colid < C
        mx = jnp.max(jnp.where(mask, logits, -jnp.inf), axis=1, keepdims=True)
        ex = jnp.where(mask, jnp.exp(logits - mx), 0.0)
        lse = jnp.log(jnp.sum(ex, axis=1, keepdims=True)) + mx
        out_ref[...] = logits - lse


_vspec = pl.BlockSpec((1, D), lambda i: (0, 0))
_wspec = pl.BlockSpec((D, D), lambda i: (0, 0))
_wespec = pl.BlockSpec((16, D), lambda i: (0, 0))
_hspec = pl.BlockSpec((BR, D), lambda i: (i, 0))
_sspec = pl.BlockSpec((2, BR, D), lambda i: (0, i, 0))
_tpqspec = pl.BlockSpec((2, BR, D), lambda i: (0, i, 0))

_tpq_call = pl.pallas_call(
    _tpq_body,
    grid=(GRID,),
    in_specs=[_hspec, _vspec, _vspec, _wespec, _vspec],
    out_specs=_tpqspec,
    out_shape=jax.ShapeDtypeStruct((2, N, D), _f32),
)

_ba_call = pl.pallas_call(
    _ba_body,
    grid=(GRID,),
    in_specs=[_sspec, _hspec, _vspec, _vspec, _wspec, _vspec,
              _vspec, _vspec, _wespec, _vspec],
    out_specs=[_hspec, _tpqspec],
    out_shape=[jax.ShapeDtypeStruct((N, D), _f32),
               jax.ShapeDtypeStruct((2, N, D), _f32)],
)

_b3_call = pl.pallas_call(
    _b3_body,
    grid=(GRID,),
    in_specs=[_sspec, _hspec, _vspec, _vspec, _wspec, _vspec,
              pl.BlockSpec((1, 1, BR), lambda i: (i, 0, 0)),
              _vspec, _vspec, _wspec, _vspec, _vspec, _vspec, _wspec, _vspec],
    out_specs=pl.BlockSpec((G, D), lambda i: (0, 0)),
    out_shape=jax.ShapeDtypeStruct((G, D), _f32),
    scratch_shapes=[pltpu.VMEM((G, D), _f32)],
)


def kernel(x, edge_index, batch, bn0_g, bn0_b, We0, be0, Wc0, bc0,
           bn1_g, bn1_b, We1, be1, Wc1, bc1, bn2_g, bn2_b, We2, be2, Wc2, bc2,
           bnfc_g, bnfc_b, Wlin, blin, bnh_g, bnh_b, Wcls, bcls):
    src = edge_index[0]
    dst = edge_index[1]
    pad = EPAD - E
    src_p = jnp.concatenate([src, jnp.zeros((pad,), jnp.int32)])
    src2 = jnp.concatenate([src_p, src_p + N])
    dstp = jnp.concatenate([dst, jnp.full((pad,), N, jnp.int32)])
    zrows = jnp.zeros((K, D), _f32)

    def v(a):
        return a.reshape(1, D)

    def we(a):
        return jnp.pad(a, ((0, 16 - a.shape[0]), (0, 0)))

    params = [
        (v(bn0_g), v(bn0_b), we(We0), v(be0), Wc0, v(bc0)),
        (v(bn1_g), v(bn1_b), we(We1), v(be1), Wc1, v(bc1)),
        (v(bn2_g), v(bn2_b), we(We2), v(be2), Wc2, v(bc2)),
    ]

    g0, b0, we0_, be0_, wc0, bc0_ = params[0]
    g1, b1, we1_, be1_, wc1, bc1_ = params[1]
    g2, b2, we2_, be2_, wc2, bc2_ = params[2]

    tpq = _tpq_call(x, g0, b0, we0_, be0_)
    s0 = _sc_edge_pass(tpq.reshape(2 * N, D), src2, dstp, zrows)
    h1, tpq = _ba_call(s0.reshape(2, N1, D), x, g0, b0, wc0, bc0_,
                       g1, b1, we1_, be1_)
    s1 = _sc_edge_pass(tpq.reshape(2 * N, D), src2, dstp, zrows)
    h2, tpq = _ba_call(s1.reshape(2, N1, D), h1, g1, b1, wc1, bc1_,
                       g2, b2, we2_, be2_)
    s2 = _sc_edge_pass(tpq.reshape(2 * N, D), src2, dstp, zrows)

    batch3 = batch.reshape(GRID, 1, BR)
    wcls_p = jnp.pad(Wcls, ((0, 0), (0, D - C)))
    bcls_p = jnp.pad(bcls, ((0, D - C))).reshape(1, D)
    out = _b3_call(s2.reshape(2, N1, D), h2, g2, b2, wc2, bc2_, batch3,
                   v(bnfc_g), v(bnfc_b), Wlin, v(blin),
                   v(bnh_g), v(bnh_b), wcls_p, bcls_p)
    return out[:, :C]


# speedup vs baseline: 10.1512x; 1.1696x over previous
import functools

import jax
import jax.numpy as jnp
from jax import lax
from jax.experimental import pallas as pl
from jax.experimental.pallas import tpu as pltpu
from jax.experimental.pallas import tpu_sc as plsc

N = 10000
E = 320000
D = 128
G = 64
C = 10

NSUB = 16
K = 128
CH = (E + NSUB * K - 1) // (NSUB * K)
CH += CH % 2
EPAD = NSUB * K * CH
N1 = 10112
RPT = N1 // NSUB

BR = 1000
GRID = N // BR

_f32 = jnp.float32


def _sc_body(tpq, src2, dstp, zrows, out, idx_s, idx_d, rows, acc, sem0, sem1):
    sem = (sem0, sem1)
    cid = lax.axis_index("c")
    sid = lax.axis_index("s")
    rbase = sid * RPT

    pltpu.sync_copy(zrows, rows.at[pl.ds(0, K)])
    for j in range((RPT + K - 1) // K):
        sz = min(K, RPT - j * K)
        pltpu.sync_copy(rows.at[pl.ds(0, sz)],
                        acc.at[pl.ds(rbase + j * K, sz)])
    plsc.subcore_barrier()

    ebase = sid * (CH * K)

    def load_start(c, slot):
        b = ebase + c * K
        pltpu.sync_copy(src2.at[pl.ds(cid * EPAD + b, K)], idx_s.at[slot])
        pltpu.sync_copy(dstp.at[pl.ds(b, K)], idx_d.at[slot])
        pltpu.async_copy(tpq.at[idx_s.at[slot]],
                         rows.at[pl.ds(slot * K, K)], sem[slot])

    def drain_scatter(slot):
        pltpu.make_async_copy(tpq.at[idx_s.at[slot]],
                              rows.at[pl.ds(slot * K, K)], sem[slot]).wait()
        pltpu.sync_copy(rows.at[pl.ds(slot * K, K)], acc.at[idx_d.at[slot]],
                        add=True)

    load_start(0, 0)

    def pair(i2, carry):
        load_start(2 * i2 + 1, 1)
        drain_scatter(0)

        @pl.when(2 * i2 + 2 < CH)
        def _():
            load_start(2 * i2 + 2, 0)

        drain_scatter(1)
        return carry

    lax.fori_loop(0, CH // 2, pair, 0)
    plsc.subcore_barrier()

    for j in range((RPT + K - 1) // K):
        sz = min(K, RPT - j * K)
        pltpu.sync_copy(acc.at[pl.ds(rbase + j * K, sz)],
                        rows.at[pl.ds(0, sz)])
        pltpu.sync_copy(rows.at[pl.ds(0, sz)],
                        out.at[pl.ds(cid * N1 + rbase + j * K, sz)])


@functools.cache
def _sc_kernel():
    return pl.kernel(
        _sc_body,
        out_type=jax.ShapeDtypeStruct((2 * N1, D), _f32),
        mesh=plsc.VectorSubcoreMesh(core_axis_name="c", subcore_axis_name="s"),
        scratch_types=[
            pltpu.VMEM((2, K), jnp.int32),
            pltpu.VMEM((2, K), jnp.int32),
            pltpu.VMEM((2 * K, D), _f32),
            pltpu.VMEM_SHARED((N1, D), _f32),
            pltpu.SemaphoreType.DMA,
            pltpu.SemaphoreType.DMA,
        ],
    )


def _sc_edge_pass(tpq2n, src2, dstp, zrows):
    return _sc_kernel()(tpq2n, src2, dstp, zrows)


def _node_m(h, g, b, we, be):
    e = we[7:8, :] + be[...]
    m = jnp.maximum(h * g[...] + b[...] + e, 0.0) + 1e-7
    return m


def _tpq_body(h_ref, g_ref, b_ref, we_ref, be_ref, tpq_ref):
    m = _node_m(h_ref[...], g_ref, b_ref, we_ref, be_ref)
    p = jnp.exp(m)
    tpq_ref[0] = p
    tpq_ref[1] = m * p


def _conv_out(s_ref, h_ref, g0, b0, wc, bc):
    agg = s_ref[1] / (s_ref[0] + 1e-30)
    hn = h_ref[...] * g0[...] + b0[...]
    z = jnp.dot(hn + agg, wc[...], preferred_element_type=_f32) + bc[...]
    return jnp.maximum(z, 0.0)


def _ba_body(s_ref, h_ref, g0, b0, wc, bc, g1, b1, we1, be1, hout_ref, tpq_ref):
    hnew = _conv_out(s_ref, h_ref, g0, b0, wc, bc)
    hout_ref[...] = hnew
    m = _node_m(hnew, g1, b1, we1, be1)
    p = jnp.exp(m)
    tpq_ref[0] = p
    tpq_ref[1] = m * p


def _b3_body(s_ref, h_ref, g2, b2, wc2, bc2, batch_ref, gfc, bfc, wlin, blin,
             gh, bh, wcls, bcls, out_ref, pooled):
    i = pl.program_id(0)
    h3 = _conv_out(s_ref, h_ref, g2, b2, wc2, bc2)
    bvec = batch_ref[0, 0, :]
    onehot = (bvec[:, None]
              == lax.broadcasted_iota(jnp.int32, (BR, G), 1)).astype(_f32)
    part = lax.dot_general(onehot, h3, (((0,), (0,)), ((), ())),
                           preferred_element_type=_f32)

    @pl.when(i == 0)
    def _():
        pooled[...] = jnp.zeros_like(pooled)

    pooled[...] += part

    @pl.when(i == GRID - 1)
    def _():
        pool = pooled[...]
        z = jnp.maximum(
            jnp.dot(pool * gfc[...] + bfc[...], wlin[...],
                    preferred_element_type=_f32) + blin[...], 0.0)
        z = z * gh[...] + bh[...]
        logits = jnp.dot(z, wcls[...], preferred_element_type=_f32) + bcls[...]
        colid = lax.broadcasted_iota(jnp.int32, (G, D), 1)
        mask = colid < C
        mx = jnp.max(jnp.where(mask, logits, -jnp.inf), axis=1, keepdims=True)
        ex = jnp.where(mask, jnp.exp(logits - mx), 0.0)
        lse = jnp.log(jnp.sum(ex, axis=1, keepdims=True)) + mx
        out_ref[...] = logits - lse


_vspec = pl.BlockSpec((1, D), lambda i: (0, 0))
_wspec = pl.BlockSpec((D, D), lambda i: (0, 0))
_wespec = pl.BlockSpec((16, D), lambda i: (0, 0))
_hspec = pl.BlockSpec((BR, D), lambda i: (i, 0))
_sspec = pl.BlockSpec((2, BR, D), lambda i: (0, i, 0))
_tpqspec = pl.BlockSpec((2, BR, D), lambda i: (0, i, 0))

_tpq_call = pl.pallas_call(
    _tpq_body,
    grid=(GRID,),
    in_specs=[_hspec, _vspec, _vspec, _wespec, _vspec],
    out_specs=_tpqspec,
    out_shape=jax.ShapeDtypeStruct((2, N, D), _f32),
)

_ba_call = pl.pallas_call(
    _ba_body,
    grid=(GRID,),
    in_specs=[_sspec, _hspec, _vspec, _vspec, _wspec, _vspec,
              _vspec, _vspec, _wespec, _vspec],
    out_specs=[_hspec, _tpqspec],
    out_shape=[jax.ShapeDtypeStruct((N, D), _f32),
               jax.ShapeDtypeStruct((2, N, D), _f32)],
)

_b3_call = pl.pallas_call(
    _b3_body,
    grid=(GRID,),
    in_specs=[_sspec, _hspec, _vspec, _vspec, _wspec, _vspec,
              pl.BlockSpec((1, 1, BR), lambda i: (i, 0, 0)),
              _vspec, _vspec, _wspec, _vspec, _vspec, _vspec, _wspec, _vspec],
    out_specs=pl.BlockSpec((G, D), lambda i: (0, 0)),
    out_shape=jax.ShapeDtypeStruct((G, D), _f32),
    scratch_shapes=[pltpu.VMEM((G, D), _f32)],
)


def kernel(x, edge_index, batch, bn0_g, bn0_b, We0, be0, Wc0, bc0,
           bn1_g, bn1_b, We1, be1, Wc1, bc1, bn2_g, bn2_b, We2, be2, Wc2, bc2,
           bnfc_g, bnfc_b, Wlin, blin, bnh_g, bnh_b, Wcls, bcls):
    src = edge_index[0]
    dst = edge_index[1]
    pad = EPAD - E
    src_p = jnp.concatenate([src, jnp.zeros((pad,), jnp.int32)])
    src2 = jnp.concatenate([src_p, src_p + N])
    dstp = jnp.concatenate([dst, jnp.full((pad,), N, jnp.int32)])
    zrows = jnp.zeros((K, D), _f32)

    def v(a):
        return a.reshape(1, D)

    def we(a):
        return jnp.pad(a, ((0, 16 - a.shape[0]), (0, 0)))

    params = [
        (v(bn0_g), v(bn0_b), we(We0), v(be0), Wc0, v(bc0)),
        (v(bn1_g), v(bn1_b), we(We1), v(be1), Wc1, v(bc1)),
        (v(bn2_g), v(bn2_b), we(We2), v(be2), Wc2, v(bc2)),
    ]

    g0, b0, we0_, be0_, wc0, bc0_ = params[0]
    g1, b1, we1_, be1_, wc1, bc1_ = params[1]
    g2, b2, we2_, be2_, wc2, bc2_ = params[2]

    tpq = _tpq_call(x, g0, b0, we0_, be0_)
    s0 = _sc_edge_pass(tpq.reshape(2 * N, D), src2, dstp, zrows)
    h1, tpq = _ba_call(s0.reshape(2, N1, D), x, g0, b0, wc0, bc0_,
                       g1, b1, we1_, be1_)
    s1 = _sc_edge_pass(tpq.reshape(2 * N, D), src2, dstp, zrows)
    h2, tpq = _ba_call(s1.reshape(2, N1, D), h1, g1, b1, wc1, bc1_,
                       g2, b2, we2_, be2_)
    s2 = _sc_edge_pass(tpq.reshape(2 * N, D), src2, dstp, zrows)

    batch3 = batch.reshape(GRID, 1, BR)
    wcls_p = jnp.pad(Wcls, ((0, 0), (0, D - C)))
    bcls_p = jnp.pad(bcls, ((0, D - C))).reshape(1, D)
    out = _b3_call(s2.reshape(2, N1, D), h2, g2, b2, wc2, bc2_, batch3,
                   v(bnfc_g), v(bnfc_b), Wlin, v(blin),
                   v(bnh_g), v(bnh_b), wcls_p, bcls_p)
    return out[:, :C]


# speedup vs baseline: 11.9762x; 1.1798x over previous
import functools

import jax
import jax.numpy as jnp
from jax import lax
from jax.experimental import pallas as pl
from jax.experimental.pallas import tpu as pltpu
from jax.experimental.pallas import tpu_sc as plsc

N = 10000
E = 320000
D = 128
G = 64
C = 10

NSUB = 16
K = 128
CH = (E + NSUB * K - 1) // (NSUB * K)
CH += CH % 2
EPAD = NSUB * K * CH
N1 = 10112
RPT = N1 // NSUB

BR = 1000
GRID = N // BR

_f32 = jnp.float32


def _sc_body(tpq, inter, zrows, out, idx2, rows, acc, sem0, sem1):
    sem = (sem0, sem1)
    cid = lax.axis_index("c")
    sid = lax.axis_index("s")
    rbase = sid * RPT

    pltpu.sync_copy(zrows, rows.at[pl.ds(0, K)])
    for j in range((RPT + K - 1) // K):
        sz = min(K, RPT - j * K)
        pltpu.sync_copy(rows.at[pl.ds(0, sz)],
                        acc.at[pl.ds(rbase + j * K, sz)])
    plsc.subcore_barrier()

    cbase = cid * (NSUB * CH) + sid * CH

    def load_start(c, slot):
        pltpu.sync_copy(inter.at[cbase + c], idx2.at[slot])
        pltpu.async_copy(tpq.at[idx2.at[slot, 0]],
                         rows.at[pl.ds(slot * K, K)], sem[slot])

    def drain_scatter(slot):
        pltpu.make_async_copy(tpq.at[idx2.at[slot, 0]],
                              rows.at[pl.ds(slot * K, K)], sem[slot]).wait()
        pltpu.sync_copy(rows.at[pl.ds(slot * K, K)], acc.at[idx2.at[slot, 1]],
                        add=True)

    load_start(0, 0)

    def pair(i2, carry):
        load_start(2 * i2 + 1, 1)
        drain_scatter(0)

        @pl.when(2 * i2 + 2 < CH)
        def _():
            load_start(2 * i2 + 2, 0)

        drain_scatter(1)
        return carry

    lax.fori_loop(0, CH // 2, pair, 0)
    plsc.subcore_barrier()

    for j in range((RPT + K - 1) // K):
        sz = min(K, RPT - j * K)
        pltpu.sync_copy(acc.at[pl.ds(rbase + j * K, sz)],
                        rows.at[pl.ds(0, sz)])
        pltpu.sync_copy(rows.at[pl.ds(0, sz)],
                        out.at[pl.ds(cid * N1 + rbase + j * K, sz)])


@functools.cache
def _sc_kernel():
    return pl.kernel(
        _sc_body,
        out_type=jax.ShapeDtypeStruct((2 * N1, D), _f32),
        mesh=plsc.VectorSubcoreMesh(core_axis_name="c", subcore_axis_name="s"),
        scratch_types=[
            pltpu.VMEM((2, 2, K), jnp.int32),
            pltpu.VMEM((2 * K, D), _f32),
            pltpu.VMEM_SHARED((N1, D), _f32),
            pltpu.SemaphoreType.DMA,
            pltpu.SemaphoreType.DMA,
        ],
    )


def _sc_edge_pass(tpq2n, inter, zrows):
    return _sc_kernel()(tpq2n, inter, zrows)


def _node_m(h, g, b, we, be):
    e = we[7:8, :] + be[...]
    m = jnp.maximum(h * g[...] + b[...] + e, 0.0) + 1e-7
    return m


def _tpq_body(h_ref, g_ref, b_ref, we_ref, be_ref, tpq_ref):
    m = _node_m(h_ref[...], g_ref, b_ref, we_ref, be_ref)
    p = jnp.exp(m)
    tpq_ref[0] = p
    tpq_ref[1] = m * p


def _conv_out(s_ref, h_ref, g0, b0, wc, bc):
    agg = s_ref[1] / (s_ref[0] + 1e-30)
    hn = h_ref[...] * g0[...] + b0[...]
    z = jnp.dot(hn + agg, wc[...], preferred_element_type=_f32) + bc[...]
    return jnp.maximum(z, 0.0)


def _ba_body(s_ref, h_ref, g0, b0, wc, bc, g1, b1, we1, be1, hout_ref, tpq_ref):
    hnew = _conv_out(s_ref, h_ref, g0, b0, wc, bc)
    hout_ref[...] = hnew
    m = _node_m(hnew, g1, b1, we1, be1)
    p = jnp.exp(m)
    tpq_ref[0] = p
    tpq_ref[1] = m * p


def _b3_body(s_ref, h_ref, g2, b2, wc2, bc2, batch_ref, gfc, bfc, wlin, blin,
             gh, bh, wcls, bcls, out_ref, pooled):
    i = pl.program_id(0)
    h3 = _conv_out(s_ref, h_ref, g2, b2, wc2, bc2)
    bvec = batch_ref[0, 0, :]
    onehot = (bvec[:, None]
              == lax.broadcasted_iota(jnp.int32, (BR, G), 1)).astype(_f32)
    part = lax.dot_general(onehot, h3, (((0,), (0,)), ((), ())),
                           preferred_element_type=_f32)

    @pl.when(i == 0)
    def _():
        pooled[...] = jnp.zeros_like(pooled)

    pooled[...] += part

    @pl.when(i == GRID - 1)
    def _():
        pool = pooled[...]
        z = jnp.maximum(
            jnp.dot(pool * gfc[...] + bfc[...], wlin[...],
                    preferred_element_type=_f32) + blin[...], 0.0)
        z = z * gh[...] + bh[...]
        logits = jnp.dot(z, wcls[...], preferred_element_type=_f32) + bcls[...]
        colid = lax.broadcasted_iota(jnp.int32, (G, D), 1)
        mask = colid < C
        mx = jnp.max(jnp.where(mask, logits, -jnp.inf), axis=1, keepdims=True)
        ex = jnp.where(mask, jnp.exp(logits - mx), 0.0)
        lse = jnp.log(jnp.sum(ex, axis=1, keepdims=True)) + mx
        out_ref[...] = logits - lse


_vspec = pl.BlockSpec((1, D), lambda i: (0, 0))
_wspec = pl.BlockSpec((D, D), lambda i: (0, 0))
_wespec = pl.BlockSpec((16, D), lambda i: (0, 0))
_hspec = pl.BlockSpec((BR, D), lambda i: (i, 0))
_sspec = pl.BlockSpec((2, BR, D), lambda i: (0, i, 0))
_tpqspec = pl.BlockSpec((2, BR, D), lambda i: (0, i, 0))

_tpq_call = pl.pallas_call(
    _tpq_body,
    grid=(GRID,),
    in_specs=[_hspec, _vspec, _vspec, _wespec, _vspec],
    out_specs=_tpqspec,
    out_shape=jax.ShapeDtypeStruct((2, N, D), _f32),
)

_ba_call = pl.pallas_call(
    _ba_body,
    grid=(GRID,),
    in_specs=[_sspec, _hspec, _vspec, _vspec, _wspec, _vspec,
              _vspec, _vspec, _wespec, _vspec],
    out_specs=[_hspec, _tpqspec],
    out_shape=[jax.ShapeDtypeStruct((N, D), _f32),
               jax.ShapeDtypeStruct((2, N, D), _f32)],
)

_b3_call = pl.pallas_call(
    _b3_body,
    grid=(GRID,),
    in_specs=[_sspec, _hspec, _vspec, _vspec, _wspec, _vspec,
              pl.BlockSpec((1, 1, BR), lambda i: (i, 0, 0)),
              _vspec, _vspec, _wspec, _vspec, _vspec, _vspec, _wspec, _vspec],
    out_specs=pl.BlockSpec((G, D), lambda i: (0, 0)),
    out_shape=jax.ShapeDtypeStruct((G, D), _f32),
    scratch_shapes=[pltpu.VMEM((G, D), _f32)],
)


def kernel(x, edge_index, batch, bn0_g, bn0_b, We0, be0, Wc0, bc0,
           bn1_g, bn1_b, We1, be1, Wc1, bc1, bn2_g, bn2_b, We2, be2, Wc2, bc2,
           bnfc_g, bnfc_b, Wlin, blin, bnh_g, bnh_b, Wcls, bcls):
    src = edge_index[0]
    dst = edge_index[1]
    pad = EPAD - E
    src_p = jnp.concatenate([src, jnp.zeros((pad,), jnp.int32)])
    dst_p = jnp.concatenate([dst, jnp.full((pad,), N, jnp.int32)])
    cs = src_p.reshape(NSUB * CH, K)
    cd = dst_p.reshape(NSUB * CH, K)
    inter = jnp.concatenate([jnp.stack([cs, cd], axis=1),
                             jnp.stack([cs + N, cd], axis=1)])
    zrows = jnp.zeros((K, D), _f32)

    def v(a):
        return a.reshape(1, D)

    def we(a):
        return jnp.pad(a, ((0, 16 - a.shape[0]), (0, 0)))

    params = [
        (v(bn0_g), v(bn0_b), we(We0), v(be0), Wc0, v(bc0)),
        (v(bn1_g), v(bn1_b), we(We1), v(be1), Wc1, v(bc1)),
        (v(bn2_g), v(bn2_b), we(We2), v(be2), Wc2, v(bc2)),
    ]

    g0, b0, we0_, be0_, wc0, bc0_ = params[0]
    g1, b1, we1_, be1_, wc1, bc1_ = params[1]
    g2, b2, we2_, be2_, wc2, bc2_ = params[2]

    tpq = _tpq_call(x, g0, b0, we0_, be0_)
    s0 = _sc_edge_pass(tpq.reshape(2 * N, D), inter, zrows)
    h1, tpq = _ba_call(s0.reshape(2, N1, D), x, g0, b0, wc0, bc0_,
                       g1, b1, we1_, be1_)
    s1 = _sc_edge_pass(tpq.reshape(2 * N, D), inter, zrows)
    h2, tpq = _ba_call(s1.reshape(2, N1, D), h1, g1, b1, wc1, bc1_,
                       g2, b2, we2_, be2_)
    s2 = _sc_edge_pass(tpq.reshape(2 * N, D), inter, zrows)

    batch3 = batch.reshape(GRID, 1, BR)
    wcls_p = jnp.pad(Wcls, ((0, 0), (0, D - C)))
    bcls_p = jnp.pad(bcls, ((0, D - C))).reshape(1, D)
    out = _b3_call(s2.reshape(2, N1, D), h2, g2, b2, wc2, bc2_, batch3,
                   v(bnfc_g), v(bnfc_b), Wlin, v(blin),
                   v(bnh_g), v(bnh_b), wcls_p, bcls_p)
    return out[:, :C]
